# baseline (speedup 1.0000x reference)
import jax
import jax.numpy as jnp
from jax import lax
from jax.experimental import pallas as pl
from jax.experimental.pallas import tpu as pltpu, tpu_sc as plsc

NC, NS, L = 2, 16, 16
NW = NC * NS


def _body(fpack_h, src_h, dst_h, zeros_h, spart_h, out_h,
          src_v, dst_v, w_v, rows_s, rows_d, out_v, s_a, s_b,
          sem_s, sem_d, xsem, s_sc):
    cid = lax.axis_index("c")
    sid = lax.axis_index("s")
    wid = sid * NC + cid
    n_sub, b = src_v.shape
    n_grp = b // L

    @pl.when(sid == 0)
    def _():
        pltpu.sync_copy(zeros_h, s_sc)
    plsc.subcore_barrier()

    pltpu.sync_copy(src_h.at[wid], src_v)
    pltpu.sync_copy(dst_h.at[wid], dst_v)

    def start_gather(c, p):
        pltpu.async_copy(fpack_h.at[src_v.at[c]], rows_s.at[p], sem_s)
        pltpu.async_copy(fpack_h.at[dst_v.at[c]], rows_d.at[p], sem_d)

    def wait_gather(c, p):
        pltpu.make_async_copy(fpack_h.at[src_v.at[c]], rows_s.at[p], sem_s).wait()
        pltpu.make_async_copy(fpack_h.at[dst_v.at[c]], rows_d.at[p], sem_d).wait()

    start_gather(0, 0)

    def sub_body(c, _):
        p = lax.rem(c, 2)
        wait_gather(c, p)

        @pl.when(c + 1 < n_sub)
        def _():
            start_gather(c + 1, 1 - p)

        rs = rows_s.at[p]
        rd = rows_d.at[p]
        n_words = rows_s.shape[2]

        def grp_body(k, _):
            r16 = lax.iota(jnp.int32, L) + k * L
            rot = lax.iota(jnp.int32, L)
            wmask = jnp.full((L,), n_words - 1, jnp.int32)

            def d_body(dc, accs):
                a0, a1, a2, a3 = accs
                for dd in range(8):
                    col = (rot + (dd + dc * 8)) & wmask
                    a = plsc.load_gather(rs, [r16, col])
                    bv = plsc.load_gather(rd, [r16, col])
                    alo, ahi = plsc.unpack(
                        plsc.bitcast(a, jnp.bfloat16),
                        format=plsc.PackFormat.INTERLEAVED)
                    blo, bhi = plsc.unpack(
                        plsc.bitcast(bv, jnp.bfloat16),
                        format=plsc.PackFormat.INTERLEAVED)
                    if dd % 2 == 0:
                        a0 = a0 + alo * blo
                        a1 = a1 + ahi * bhi
                    else:
                        a2 = a2 + alo * blo
                        a3 = a3 + ahi * bhi
                return (a0, a1, a2, a3)

            z = jnp.zeros((L,), jnp.float32)
            accs = lax.fori_loop(0, n_words // 8, d_body, (z, z, z, z))
            dotp = (accs[0] + accs[1]) + (accs[2] + accs[3])
            w16 = jnp.exp(jnp.exp(jnp.abs(dotp) * (-0.01)))
            w_v[c, pl.ds(k * L, L)] = w16
            return 0

        lax.fori_loop(0, n_grp, grp_body, 0)
        return 0

    lax.fori_loop(0, n_sub, sub_body, 0)

    def scat_body(c, _):
        pltpu.sync_copy(w_v.at[c], s_sc.at[dst_v.at[c]], add=True)
        return 0

    lax.fori_loop(0, n_sub, scat_body, 0)
    plsc.subcore_barrier()

    @pl.when(sid == 0)
    def _():
        @pl.when(cid == 1)
        def _():
            def spin(i, acc):
                idx = acc & 63
                v = dst_v[0, pl.ds(idx, L)]
                return acc + jnp.max(v) + 1
            acc = lax.fori_loop(0, 20000, spin, 0)
            w_v[0, pl.ds(0, L)] = w_v[0, pl.ds(0, L)] + (acc == -1).astype(jnp.float32)
        pltpu.sync_copy(s_sc, spart_h.at[cid])
        pltpu.semaphore_signal(xsem, 1, core_index=1 - cid)
        pltpu.semaphore_wait(xsem, 1)
    plsc.subcore_barrier()

    pltpu.sync_copy(spart_h.at[0], s_a)
    pltpu.sync_copy(spart_h.at[1], s_b)
    n_nodes = s_a.shape[0]

    def sum_body(i, _):
        sl = pl.ds(i * L, L)
        s_a[sl] = s_a[sl] + s_b[sl]
        return 0

    lax.fori_loop(0, n_nodes // L, sum_body, 0)

    def div_body(c, _):
        def grp_body(k, _):
            sl = pl.ds(k * L, L)
            d16 = dst_v[c, sl]
            s16 = plsc.load_gather(s_a, [d16])
            out_v[c, sl] = w_v[c, sl] / s16
            return 0

        lax.fori_loop(0, n_grp, grp_body, 0)
        return 0

    lax.fori_loop(0, n_sub, div_body, 0)
    pltpu.sync_copy(out_v, out_h.at[wid])


def kernel(feats, edge_index):
    n_nodes, d_feat = feats.shape
    e = edge_index.shape[1]
    chunk = e // NW
    b = 80
    n_sub = chunk // b

    src3 = edge_index[0].reshape(NW, n_sub, b)
    dst3 = edge_index[1].reshape(NW, n_sub, b)
    zeros = jnp.zeros((n_nodes,), jnp.float32)
    fpack = jax.lax.bitcast_convert_type(
        feats.astype(jnp.bfloat16).reshape(n_nodes, d_feat // 2, 2),
        jnp.int32)

    mesh = plsc.VectorSubcoreMesh(core_axis_name="c", subcore_axis_name="s",
                                  num_cores=NC, num_subcores=NS)
    cparams = pltpu.CompilerParams(needs_layout_passes=False,
                                   use_tc_tiling_on_sc=False)

    k1 = pl.kernel(
        _body,
        out_type=[
            jax.ShapeDtypeStruct((NC, n_nodes), jnp.float32),
            jax.ShapeDtypeStruct((NW, n_sub, b), jnp.float32),
        ],
        mesh=mesh,
        compiler_params=cparams,
        scratch_types=[
            pltpu.VMEM((n_sub, b), jnp.int32),
            pltpu.VMEM((n_sub, b), jnp.int32),
            pltpu.VMEM((n_sub, b), jnp.float32),
            pltpu.VMEM((2, b, d_feat // 2), jnp.int32),
            pltpu.VMEM((2, b, d_feat // 2), jnp.int32),
            pltpu.VMEM((n_sub, b), jnp.float32),
            pltpu.VMEM((n_nodes,), jnp.float32),
            pltpu.VMEM((n_nodes,), jnp.float32),
            pltpu.SemaphoreType.DMA,
            pltpu.SemaphoreType.DMA,
            pltpu.SemaphoreType.REGULAR,
            pltpu.VMEM_SHARED((n_nodes,), jnp.float32),
        ],
    )
    _, out3 = k1(fpack, src3, dst3, zeros)
    return out3.reshape(e, 1)


# speedup vs baseline: 2.7336x; 2.7336x over previous
import jax
import jax.numpy as jnp
from jax import lax
from jax.experimental import pallas as pl
from jax.experimental.pallas import tpu as pltpu, tpu_sc as plsc

NC, NS, L = 2, 16, 16
NW = NC * NS


def _body(fpack_h, src_h, dst_h, zeros_h, spart_h, out_h,
          src_v, dst_v, w_v, rows_s, rows_d, out_v, s_a, s_b,
          sem_s, sem_d, sem_w, xsem, s_sc):
    cid = lax.axis_index("c")
    sid = lax.axis_index("s")
    wid = sid * NC + cid
    n_sub, b = src_v.shape
    n_grp = b // L

    @pl.when(sid == 0)
    def _():
        pltpu.sync_copy(zeros_h, s_sc)
    plsc.subcore_barrier()

    pltpu.sync_copy(src_h.at[wid], src_v)
    pltpu.sync_copy(dst_h.at[wid], dst_v)

    def start_gather(c, p):
        pltpu.async_copy(fpack_h.at[src_v.at[c]], rows_s.at[p], sem_s)
        pltpu.async_copy(fpack_h.at[dst_v.at[c]], rows_d.at[p], sem_d)

    def wait_gather(c, p):
        pltpu.make_async_copy(fpack_h.at[src_v.at[c]], rows_s.at[p], sem_s).wait()
        pltpu.make_async_copy(fpack_h.at[dst_v.at[c]], rows_d.at[p], sem_d).wait()

    start_gather(0, 0)

    def sub_body(c, _):
        p = lax.rem(c, 2)
        wait_gather(c, p)

        @pl.when(c + 1 < n_sub)
        def _():
            start_gather(c + 1, 1 - p)

        rs = rows_s.at[p]
        rd = rows_d.at[p]
        n_words = rows_s.shape[2]

        def grp_body(k, _):
            r16 = lax.iota(jnp.int32, L) + k * L
            rot = lax.iota(jnp.int32, L)
            wmask = jnp.full((L,), n_words - 1, jnp.int32)

            def d_body(dc, accs):
                a0, a1, a2, a3 = accs
                for dd in range(8):
                    col = (rot + (dd + dc * 8)) & wmask
                    a = plsc.load_gather(rs, [r16, col])
                    bv = plsc.load_gather(rd, [r16, col])
                    prod = (plsc.bitcast(a, jnp.bfloat16) *
                            plsc.bitcast(bv, jnp.bfloat16))
                    plo, phi = plsc.unpack(
                        prod, format=plsc.PackFormat.INTERLEAVED)
                    if dd % 2 == 0:
                        a0 = a0 + plo
                        a1 = a1 + phi
                    else:
                        a2 = a2 + plo
                        a3 = a3 + phi
                return (a0, a1, a2, a3)

            z = jnp.zeros((L,), jnp.float32)
            accs = lax.fori_loop(0, n_words // 8, d_body, (z, z, z, z))
            dotp = (accs[0] + accs[1]) + (accs[2] + accs[3])
            w16 = jnp.exp(jnp.exp(jnp.abs(dotp) * (-0.01)))
            w_v[c, pl.ds(k * L, L)] = w16
            return 0

        lax.fori_loop(0, n_grp, grp_body, 0)
        return 0

    lax.fori_loop(0, n_sub, sub_body, 0)

    def scat_body(c, _):
        pltpu.async_copy(w_v.at[c], s_sc.at[dst_v.at[c]], sem_w, add=True)
        return 0

    lax.fori_loop(0, n_sub, scat_body, 0)

    def drain_body(c, _):
        pltpu.make_async_copy(
            w_v.at[c], s_sc.at[dst_v.at[c]], sem_w).wait()
        return 0

    lax.fori_loop(0, n_sub, drain_body, 0)
    plsc.subcore_barrier()

    @pl.when(sid == 0)
    def _():
        pltpu.sync_copy(s_sc, spart_h.at[cid])
        pltpu.semaphore_signal(xsem, 1, core_index=1 - cid)
        pltpu.semaphore_wait(xsem, 1)
    plsc.subcore_barrier()

    pltpu.sync_copy(spart_h.at[0], s_a)
    pltpu.sync_copy(spart_h.at[1], s_b)
    n_nodes = s_a.shape[0]

    def sum_body(i, _):
        sl = pl.ds(i * L, L)
        s_a[sl] = s_a[sl] + s_b[sl]
        return 0

    lax.fori_loop(0, n_nodes // L, sum_body, 0)

    def div_body(c, _):
        def grp_body(k, _):
            sl = pl.ds(k * L, L)
            d16 = dst_v[c, sl]
            s16 = plsc.load_gather(s_a, [d16])
            out_v[c, sl] = w_v[c, sl] / s16
            return 0

        lax.fori_loop(0, n_grp, grp_body, 0)
        return 0

    lax.fori_loop(0, n_sub, div_body, 0)
    pltpu.sync_copy(out_v, out_h.at[wid])


def kernel(feats, edge_index):
    n_nodes, d_feat = feats.shape
    e = edge_index.shape[1]
    chunk = e // NW
    b = 80
    n_sub = chunk // b

    src3 = edge_index[0].reshape(NW, n_sub, b)
    dst3 = edge_index[1].reshape(NW, n_sub, b)
    zeros = jnp.zeros((n_nodes,), jnp.float32)
    fpack = jax.lax.bitcast_convert_type(
        feats.astype(jnp.bfloat16).reshape(n_nodes, d_feat // 2, 2),
        jnp.int32)

    mesh = plsc.VectorSubcoreMesh(core_axis_name="c", subcore_axis_name="s",
                                  num_cores=NC, num_subcores=NS)
    cparams = pltpu.CompilerParams(needs_layout_passes=False,
                                   use_tc_tiling_on_sc=False)

    k1 = pl.kernel(
        _body,
        out_type=[
            jax.ShapeDtypeStruct((NC, n_nodes), jnp.float32),
            jax.ShapeDtypeStruct((NW, n_sub, b), jnp.float32),
        ],
        mesh=mesh,
        compiler_params=cparams,
        scratch_types=[
            pltpu.VMEM((n_sub, b), jnp.int32),
            pltpu.VMEM((n_sub, b), jnp.int32),
            pltpu.VMEM((n_sub, b), jnp.float32),
            pltpu.VMEM((2, b, d_feat // 2), jnp.int32),
            pltpu.VMEM((2, b, d_feat // 2), jnp.int32),
            pltpu.VMEM((n_sub, b), jnp.float32),
            pltpu.VMEM((n_nodes,), jnp.float32),
            pltpu.VMEM((n_nodes,), jnp.float32),
            pltpu.SemaphoreType.DMA,
            pltpu.SemaphoreType.DMA,
            pltpu.SemaphoreType.DMA,
            pltpu.SemaphoreType.REGULAR,
            pltpu.VMEM_SHARED((n_nodes,), jnp.float32),
        ],
    )
    _, out3 = k1(fpack, src3, dst3, zeros)
    return out3.reshape(e, 1)
